# Initial kernel scaffold; baseline (speedup 1.0000x reference)
import jax
import jax.numpy as jnp
from jax.experimental import pallas as pl


def kernel(glyph_ids, weight):
    raise NotImplementedError("write your pallas kernel here")


# speedup vs baseline: 1.2833x; 1.2833x over previous
import jax
import jax.numpy as jnp
from jax import lax
from jax.experimental import pallas as pl
from jax.experimental.pallas import tpu as pltpu
from jax.experimental.pallas import tpu_sc as plsc

EMBED_D = 32
NC, NS = 2, 16
NW = NC * NS
GPC = 128
G = 8
C_GROUPS = G


def _body(table_hbm, idx_hbm, out_hbm, idx_v, rows_v, gsem):
    wid = lax.axis_index("s") * NC + lax.axis_index("c")
    n_groups = idx_hbm.shape[0]
    groups_per_w = n_groups // NW
    chunks = groups_per_w // G
    base_grp = wid * groups_per_w

    def chunk_body(i, carry):
        grp = base_grp + i * G
        pltpu.sync_copy(idx_hbm.at[pl.ds(grp, G)], idx_v)
        cps = [
            pltpu.async_copy(table_hbm.at[idx_v.at[g]], rows_v.at[g], gsem)
            for g in range(G)
        ]
        for cp in cps:
            cp.wait()
        pltpu.sync_copy(rows_v, out_hbm.at[pl.ds(grp, G)])
        return carry

    lax.fori_loop(0, chunks, chunk_body, 0)


@jax.jit
def kernel(glyph_ids, weight):
    S0, S1 = glyph_ids.shape
    B = S0 * S1
    idx = glyph_ids.reshape(B // GPC, GPC).astype(jnp.int32)
    mesh = plsc.VectorSubcoreMesh(
        core_axis_name="c", subcore_axis_name="s",
        num_cores=NC, num_subcores=NS,
    )
    out = pl.kernel(
        _body,
        out_type=jax.ShapeDtypeStruct((B // GPC, GPC, EMBED_D), jnp.float32),
        mesh=mesh,
        scratch_types=[
            pltpu.VMEM((G, GPC), jnp.int32),
            pltpu.VMEM((G, GPC, EMBED_D), jnp.float32),
            pltpu.SemaphoreType.DMA,
        ],
        compiler_params=pltpu.CompilerParams(use_tc_tiling_on_sc=False),
    )(weight, idx)
    return out.reshape(S0, S1, EMBED_D)


# speedup vs baseline: 1.3099x; 1.0207x over previous
import jax
import jax.numpy as jnp
from jax import lax
from jax.experimental import pallas as pl
from jax.experimental.pallas import tpu as pltpu
from jax.experimental.pallas import tpu_sc as plsc

EMBED_D = 32
NC, NS = 2, 16
NW = NC * NS
GPC = 128
G = 10


def _body(table_hbm, idx_hbm, out_hbm,
          idx0, idx1, rows0, rows1,
          isem0, isem1, gsem0, gsem1, ssem0, ssem1):
    idx_v = (idx0, idx1)
    rows_v = (rows0, rows1)
    isem = (isem0, isem1)
    gsem = (gsem0, gsem1)
    ssem = (ssem0, ssem1)

    wid = lax.axis_index("s") * NC + lax.axis_index("c")
    n_groups = idx_hbm.shape[0]
    groups_per_w = n_groups // NW
    chunks = groups_per_w // G
    base_grp = wid * groups_per_w

    def ld_idx(c, b):
        pltpu.async_copy(idx_hbm.at[pl.ds(base_grp + c * G, G)],
                         idx_v[b], isem[b])

    def wait_idx(b):
        pltpu.make_async_copy(idx_hbm.at[pl.ds(0, G)], idx_v[b],
                              isem[b]).wait()

    def fire_gathers(b):
        for g in range(G):
            pltpu.async_copy(table_hbm.at[idx_v[b].at[g]],
                             rows_v[b].at[g], gsem[b])

    def wait_gathers(b):
        pltpu.make_async_copy(out_hbm.at[pl.ds(0, G)], rows_v[b],
                              gsem[b]).wait()

    def st_rows(c, b):
        pltpu.async_copy(rows_v[b], out_hbm.at[pl.ds(base_grp + c * G, G)],
                         ssem[b])

    def wait_store(b):
        pltpu.make_async_copy(rows_v[b], out_hbm.at[pl.ds(0, G)],
                              ssem[b]).wait()

    ld_idx(0, 0)
    ld_idx(1, 1)
    wait_idx(0)
    fire_gathers(0)
    wait_idx(1)
    fire_gathers(1)
    wait_gathers(0)
    st_rows(0, 0)
    ld_idx(2, 0)

    def step(c, b):
        wait_idx(b)
        wait_store(b)
        fire_gathers(b)
        wait_gathers(1 - b)
        st_rows(c - 1, 1 - b)
        ld_idx(jnp.minimum(c + 1, chunks - 1), 1 - b)

    def pair(q, carry):
        c0 = 2 + 2 * q
        step(c0, 0)
        step(c0 + 1, 1)
        return carry

    lax.fori_loop(0, (chunks - 2) // 2, pair, 0)

    wait_gathers(1)
    st_rows(chunks - 1, 1)
    wait_idx(0)
    wait_store(0)
    wait_store(1)


@jax.jit
def kernel(glyph_ids, weight):
    S0, S1 = glyph_ids.shape
    B = S0 * S1
    idx = glyph_ids.reshape(B // GPC, GPC).astype(jnp.int32)
    mesh = plsc.VectorSubcoreMesh(
        core_axis_name="c", subcore_axis_name="s",
        num_cores=NC, num_subcores=NS,
    )
    out = pl.kernel(
        _body,
        out_type=jax.ShapeDtypeStruct((B // GPC, GPC, EMBED_D), jnp.float32),
        mesh=mesh,
        scratch_types=[
            pltpu.VMEM((G, GPC), jnp.int32),
            pltpu.VMEM((G, GPC), jnp.int32),
            pltpu.VMEM((G, GPC, EMBED_D), jnp.float32),
            pltpu.VMEM((G, GPC, EMBED_D), jnp.float32),
            pltpu.SemaphoreType.DMA,
            pltpu.SemaphoreType.DMA,
            pltpu.SemaphoreType.DMA,
            pltpu.SemaphoreType.DMA,
            pltpu.SemaphoreType.DMA,
            pltpu.SemaphoreType.DMA,
        ],
        compiler_params=pltpu.CompilerParams(use_tc_tiling_on_sc=False),
    )(weight, idx)
    return out.reshape(S0, S1, EMBED_D)


# speedup vs baseline: 1.8037x; 1.3769x over previous
import jax
import jax.numpy as jnp
from jax import lax
from jax.experimental import pallas as pl
from jax.experimental.pallas import tpu as pltpu
from jax.experimental.pallas import tpu_sc as plsc

NC, NS = 2, 16
NW = NC * NS
R = 16


def _body(table_hbm, idx_hbm, out_hbm,
          idx0, idx1, rows0, rows1,
          isem0, isem1, gsem0, gsem1, ssem0, ssem1):
    idx_v = (idx0, idx1)
    rows_v = (rows0, rows1)
    isem = (isem0, isem1)
    gsem = (gsem0, gsem1)
    ssem = (ssem0, ssem1)

    wid = lax.axis_index("s") * NC + lax.axis_index("c")
    n_rows = idx_hbm.shape[0]
    rows_per_w = n_rows // NW
    chunks = rows_per_w // R
    base_row = wid * rows_per_w

    def ld_idx(c, b):
        pltpu.async_copy(idx_hbm.at[pl.ds(base_row + c * R, R)],
                         idx_v[b], isem[b])

    def wait_idx(b):
        pltpu.make_async_copy(idx_hbm.at[pl.ds(0, R)], idx_v[b],
                              isem[b]).wait()

    def fire_gathers(b):
        for rr in range(R):
            pltpu.async_copy(table_hbm.at[idx_v[b].at[rr]],
                             rows_v[b].at[rr], gsem[b])

    def wait_gathers(b):
        pltpu.make_async_copy(out_hbm.at[pl.ds(0, R)], rows_v[b],
                              gsem[b]).wait()

    def st_rows(c, b):
        pltpu.async_copy(rows_v[b], out_hbm.at[pl.ds(base_row + c * R, R)],
                         ssem[b])

    def wait_store(b):
        pltpu.make_async_copy(rows_v[b], out_hbm.at[pl.ds(0, R)],
                              ssem[b]).wait()

    ld_idx(0, 0)
    ld_idx(1, 1)
    wait_idx(0)
    fire_gathers(0)
    wait_idx(1)
    fire_gathers(1)
    wait_gathers(0)
    st_rows(0, 0)
    ld_idx(2, 0)

    def step(c, b):
        wait_idx(b)
        wait_store(b)
        fire_gathers(b)
        wait_gathers(1 - b)
        st_rows(c - 1, 1 - b)
        ld_idx(jnp.minimum(c + 1, chunks - 1), 1 - b)

    def pair(q, carry):
        c0 = 2 + 2 * q
        step(c0, 0)
        step(c0 + 1, 1)
        return carry

    lax.fori_loop(0, (chunks - 2) // 2, pair, 0)

    wait_gathers(1)
    st_rows(chunks - 1, 1)
    wait_idx(0)
    wait_store(0)
    wait_store(1)


@jax.jit
def kernel(glyph_ids, weight):
    S0, S1 = glyph_ids.shape
    D = weight.shape[1]
    if glyph_ids.dtype != jnp.int32:
        glyph_ids = glyph_ids.astype(jnp.int32)
    mesh = plsc.VectorSubcoreMesh(
        core_axis_name="c", subcore_axis_name="s",
        num_cores=NC, num_subcores=NS,
    )
    return pl.kernel(
        _body,
        out_type=jax.ShapeDtypeStruct((S0, S1, D), jnp.float32),
        mesh=mesh,
        scratch_types=[
            pltpu.VMEM((R, S1), jnp.int32),
            pltpu.VMEM((R, S1), jnp.int32),
            pltpu.VMEM((R, S1, D), jnp.float32),
            pltpu.VMEM((R, S1, D), jnp.float32),
            pltpu.SemaphoreType.DMA,
            pltpu.SemaphoreType.DMA,
            pltpu.SemaphoreType.DMA,
            pltpu.SemaphoreType.DMA,
            pltpu.SemaphoreType.DMA,
            pltpu.SemaphoreType.DMA,
        ],
        compiler_params=pltpu.CompilerParams(use_tc_tiling_on_sc=False),
    )(weight, glyph_ids)
